# baseline (speedup 1.0000x reference)
import functools

import jax
import jax.numpy as jnp
from jax import lax
from jax.experimental import pallas as pl
from jax.experimental.pallas import tpu as pltpu
from jax.experimental.pallas import tpu_sc as plsc

N_SRC = 10000
N_DST = 10000
E = 320000
D = 128

NC = 2
NS = 16
L = 16
NW = NC * NS
RPT = 320
TRASH = RPT
ROWS = RPT + L
CH = 2560
NCH = E // CH
FG = D // L
SB = 4


def _sc_body(xs, xd, srci, dsti, out,
             dstbuf, srcbuf, midx, msrc, asum, amax, acnt,
             rowbuf, meanbuf, maxbuf, xdbuf, sem_i, sem_r):
  wid = lax.axis_index("s") * NC + lax.axis_index("c")
  lo = wid * RPT
  nrows = jnp.minimum(N_DST - lo, RPT)
  nrows_u = nrows.astype(jnp.uint32)
  zi = jnp.zeros((L,), jnp.int32)
  zf = jnp.zeros((L,), jnp.float32)
  ninf = jnp.full((L,), -jnp.inf, jnp.float32)
  ones = jnp.ones((L,), jnp.float32)
  trashv = jnp.full((L,), TRASH, jnp.int32)

  def init_row(r, c):
    for f in range(FG):
      asum[r, pl.ds(f * L, L)] = zf
      amax[r, pl.ds(f * L, L)] = ninf
    return c
  lax.fori_loop(0, ROWS, init_row, 0)

  def init_cnt(i, c):
    acnt[pl.ds(i * L, L)] = zf
    return c
  lax.fori_loop(0, ROWS // L, init_cnt, 0)

  def init_msrc(i, c):
    msrc[pl.ds(i * L, L)] = zi
    return c
  lax.fori_loop(0, (CH + SB * L) // L, init_msrc, 0)

  def fire_idx(c, b):
    pltpu.async_copy(dsti.at[pl.ds(c * CH, CH)], dstbuf.at[b], sem_i)
    pltpu.async_copy(srci.at[pl.ds(c * CH, CH)], srcbuf.at[b], sem_i)

  def wait_idx(b):
    pltpu.make_async_copy(dsti.at[pl.ds(0, CH)], dstbuf.at[b], sem_i).wait()
    pltpu.make_async_copy(srci.at[pl.ds(0, CH)], srcbuf.at[b], sem_i).wait()

  def fire_g(g, gb):
    idxv = msrc[pl.ds(g * L, L)]
    pltpu.async_copy(xs.at[idxv], rowbuf.at[gb], sem_r)

  def drain_g(gb):
    pltpu.make_async_copy(xs.at[zi], rowbuf.at[gb], sem_r).wait()

  fire_idx(0, 0)

  def chunk_body(c, carry):
    cb = c & 1
    wait_idx(cb)

    @pl.when(c + 1 < NCH)
    def _():
      fire_idx(c + 1, 1 - cb)

    def scan_body(i, n):
      dvec = dstbuf[cb, pl.ds(i * L, L)]
      m = (dvec >= lo) & (dvec < lo + nrows)
      k = plsc.all_reduce_population_count(m)[0]

      @pl.when(k > 0)
      def _():
        svec = srcbuf[cb, pl.ds(i * L, L)]
        cs = plsc.cumsum(m.astype(jnp.int32))
        pos = n + cs - 1
        plsc.store_scatter(midx, [pos], dvec - lo, mask=m)
        plsc.store_scatter(msrc, [pos], svec, mask=m)

      return n + k

    n = lax.fori_loop(0, CH // L, scan_body, jnp.int32(0))
    midx[pl.ds(n, L)] = trashv
    ng = (n + L - 1) // L

    @pl.when(ng > 0)
    def _():
      fire_g(0, 0)

    def g_body(g, carry2):
      gb = g & 1
      drain_g(gb)

      @pl.when(g + 1 < ng)
      def _():
        fire_g(g + 1, 1 - gb)

      dvec = midx[pl.ds(g * L, L)]
      plsc.addupdate_scatter(acnt, [dvec], ones)
      for j in range(L):
        d = dvec[j]
        for f in range(FG):
          v = rowbuf[gb, j, pl.ds(f * L, L)]
          plsc.addupdate(asum.at[d, pl.ds(f * L, L)], v)
          amax[d, pl.ds(f * L, L)] = jnp.maximum(
              amax[d, pl.ds(f * L, L)], v)
      return carry2

    lax.fori_loop(0, ng, g_body, 0)
    return carry

  lax.fori_loop(0, NCH, chunk_body, 0)

  def fin_body(b, carry):
    r0 = b * L
    cvec = acnt[pl.ds(r0, L)]
    rvec = 1.0 / jnp.maximum(cvec, 1.0)
    pltpu.sync_copy(xd.at[pl.ds(lo + r0, L)], xdbuf)
    for j in range(L):
      cj = cvec[j]
      rj = rvec[j]
      for f in range(FG):
        s = asum[r0 + j, pl.ds(f * L, L)]
        meanbuf[j, pl.ds(f * L, L)] = s * rj
        mx = amax[r0 + j, pl.ds(f * L, L)]
        maxbuf[j, pl.ds(f * L, L)] = jnp.where(cj > 0.0, mx, zf)
    pltpu.sync_copy(xdbuf, out.at[pl.ds(lo + r0, L), pl.ds(0, D)])
    pltpu.sync_copy(meanbuf, out.at[pl.ds(lo + r0, L), pl.ds(D, D)])
    pltpu.sync_copy(maxbuf, out.at[pl.ds(lo + r0, L), pl.ds(2 * D, D)])
    return carry

  lax.fori_loop(0, nrows // L, fin_body, 0)


_sc_kernel = functools.partial(
    pl.kernel,
    out_type=jax.ShapeDtypeStruct((N_DST, 3 * D), jnp.float32),
    mesh=plsc.VectorSubcoreMesh(
        core_axis_name="c", subcore_axis_name="s",
        num_cores=NC, num_subcores=NS),
    compiler_params=pltpu.CompilerParams(needs_layout_passes=False),
    scratch_types=[
        pltpu.VMEM((2, CH), jnp.int32),
        pltpu.VMEM((2, CH), jnp.int32),
        pltpu.VMEM((CH + SB * L,), jnp.int32),
        pltpu.VMEM((CH + SB * L,), jnp.int32),
        pltpu.VMEM((ROWS, D), jnp.float32),
        pltpu.VMEM((ROWS, D), jnp.float32),
        pltpu.VMEM((ROWS,), jnp.float32),
        pltpu.VMEM((2, L, D), jnp.float32),
        pltpu.VMEM((L, D), jnp.float32),
        pltpu.VMEM((L, D), jnp.float32),
        pltpu.VMEM((L, D), jnp.float32),
        pltpu.SemaphoreType.DMA,
        pltpu.SemaphoreType.DMA,
    ],
)(_sc_body)


def kernel(x_src, x_dst, edge_index):
  return _sc_kernel(x_src, x_dst, edge_index[0], edge_index[1])


# speedup vs baseline: 1.2408x; 1.2408x over previous
import functools

import jax
import jax.numpy as jnp
from jax import lax
from jax.experimental import pallas as pl
from jax.experimental.pallas import tpu as pltpu
from jax.experimental.pallas import tpu_sc as plsc

N_SRC = 10000
N_DST = 10000
E = 320000
D = 128

NC = 2
NS = 16
L = 16
NW = NC * NS
RPT = 320
TRASH = RPT
ROWS = RPT + L
CH = 2560
NCH = E // CH
FG = D // L
SB = 4


def _sc_body(xs, xd, srci, dsti, out,
             dstbuf, srcbuf, midx, msrc, asum, amax, acnt,
             rowbuf, meanbuf, maxbuf, xdbuf, sem_i, sem_r):
  wid = lax.axis_index("s") * NC + lax.axis_index("c")
  lo = wid * RPT
  nrows = jnp.minimum(N_DST - lo, RPT)
  nrows_u = nrows.astype(jnp.uint32)
  zi = jnp.zeros((L,), jnp.int32)
  zf = jnp.zeros((L,), jnp.float32)
  ninf = jnp.full((L,), -jnp.inf, jnp.float32)
  ones = jnp.ones((L,), jnp.float32)
  trashv = jnp.full((L,), TRASH, jnp.int32)

  def init_row(r, c):
    for f in range(FG):
      asum[r, pl.ds(f * L, L)] = zf
      amax[r, pl.ds(f * L, L)] = ninf
    return c
  lax.fori_loop(0, ROWS, init_row, 0)

  def init_cnt(i, c):
    acnt[pl.ds(i * L, L)] = zf
    return c
  lax.fori_loop(0, ROWS // L, init_cnt, 0)

  def init_msrc(i, c):
    msrc[pl.ds(i * L, L)] = zi
    return c
  lax.fori_loop(0, (CH + SB * L) // L, init_msrc, 0)

  def fire_idx(c, b):
    pltpu.async_copy(dsti.at[pl.ds(c * CH, CH)], dstbuf.at[b], sem_i)
    pltpu.async_copy(srci.at[pl.ds(c * CH, CH)], srcbuf.at[b], sem_i)

  def wait_idx(b):
    pltpu.make_async_copy(dsti.at[pl.ds(0, CH)], dstbuf.at[b], sem_i).wait()
    pltpu.make_async_copy(srci.at[pl.ds(0, CH)], srcbuf.at[b], sem_i).wait()

  def fire_g(g, gb):
    idxv = msrc[pl.ds(g * L, L)]
    pltpu.async_copy(xs.at[idxv], rowbuf.at[gb], sem_r)

  def drain_g(gb):
    pltpu.make_async_copy(xs.at[zi], rowbuf.at[gb], sem_r).wait()

  fire_idx(0, 0)

  def chunk_body(c, carry):
    cb = c & 1
    wait_idx(cb)

    @pl.when(c + 1 < NCH)
    def _():
      fire_idx(c + 1, 1 - cb)

    def scan_body(i, n):
      dvec = dstbuf[cb, pl.ds(i * L, L)]
      svec = srcbuf[cb, pl.ds(i * L, L)]
      m = (dvec >= lo) & (dvec < lo + nrows)
      cs = plsc.cumsum(m.astype(jnp.int32))
      pos = n + cs - 1
      plsc.store_scatter(midx, [pos], dvec - lo, mask=m)
      plsc.store_scatter(msrc, [pos], svec, mask=m)
      return n + cs[L - 1]

    n = lax.fori_loop(0, CH // L, scan_body, jnp.int32(0))
    midx[pl.ds(n, L)] = trashv
    ng = (n + L - 1) // L

    @pl.when(ng > 0)
    def _():
      fire_g(0, 0)

    def g_body(g, carry2):
      gb = g & 1
      drain_g(gb)

      @pl.when(g + 1 < ng)
      def _():
        fire_g(g + 1, 1 - gb)

      dvec = midx[pl.ds(g * L, L)]
      plsc.addupdate_scatter(acnt, [dvec], ones)
      for j in range(L):
        d = dvec[j]
        for f in range(FG):
          v = rowbuf[gb, j, pl.ds(f * L, L)]
          plsc.addupdate(asum.at[d, pl.ds(f * L, L)], v)
          amax[d, pl.ds(f * L, L)] = jnp.maximum(
              amax[d, pl.ds(f * L, L)], v)
      return carry2

    lax.fori_loop(0, ng, g_body, 0)
    return carry

  lax.fori_loop(0, NCH, chunk_body, 0)

  def fin_body(b, carry):
    r0 = b * L
    cvec = acnt[pl.ds(r0, L)]
    rvec = 1.0 / jnp.maximum(cvec, 1.0)
    pltpu.sync_copy(xd.at[pl.ds(lo + r0, L)], xdbuf)
    for j in range(L):
      cj = cvec[j]
      rj = rvec[j]
      for f in range(FG):
        s = asum[r0 + j, pl.ds(f * L, L)]
        meanbuf[j, pl.ds(f * L, L)] = s * rj
        mx = amax[r0 + j, pl.ds(f * L, L)]
        maxbuf[j, pl.ds(f * L, L)] = jnp.where(cj > 0.0, mx, zf)
    pltpu.sync_copy(xdbuf, out.at[pl.ds(lo + r0, L), pl.ds(0, D)])
    pltpu.sync_copy(meanbuf, out.at[pl.ds(lo + r0, L), pl.ds(D, D)])
    pltpu.sync_copy(maxbuf, out.at[pl.ds(lo + r0, L), pl.ds(2 * D, D)])
    return carry

  lax.fori_loop(0, nrows // L, fin_body, 0)


_sc_kernel = functools.partial(
    pl.kernel,
    out_type=jax.ShapeDtypeStruct((N_DST, 3 * D), jnp.float32),
    mesh=plsc.VectorSubcoreMesh(
        core_axis_name="c", subcore_axis_name="s",
        num_cores=NC, num_subcores=NS),
    compiler_params=pltpu.CompilerParams(needs_layout_passes=False),
    scratch_types=[
        pltpu.VMEM((2, CH), jnp.int32),
        pltpu.VMEM((2, CH), jnp.int32),
        pltpu.VMEM((CH + SB * L,), jnp.int32),
        pltpu.VMEM((CH + SB * L,), jnp.int32),
        pltpu.VMEM((ROWS, D), jnp.float32),
        pltpu.VMEM((ROWS, D), jnp.float32),
        pltpu.VMEM((ROWS,), jnp.float32),
        pltpu.VMEM((2, L, D), jnp.float32),
        pltpu.VMEM((L, D), jnp.float32),
        pltpu.VMEM((L, D), jnp.float32),
        pltpu.VMEM((L, D), jnp.float32),
        pltpu.SemaphoreType.DMA,
        pltpu.SemaphoreType.DMA,
    ],
)(_sc_body)


def kernel(x_src, x_dst, edge_index):
  return _sc_kernel(x_src, x_dst, edge_index[0], edge_index[1])


# speedup vs baseline: 1.5971x; 1.2871x over previous
import functools

import jax
import jax.numpy as jnp
from jax import lax
from jax.experimental import pallas as pl
from jax.experimental.pallas import tpu as pltpu
from jax.experimental.pallas import tpu_sc as plsc

N_SRC = 10000
N_DST = 10000
E = 320000
D = 128

NC = 2
NS = 16
L = 16
NW = NC * NS
RPT = 320
TRASH = RPT
ROWS = RPT + L
CH = 2560
NCH = E // CH
FG = D // L
SB = 4


def _sc_body(xs, xd, srci, dsti, out,
             dstbuf, srcbuf, midx, msrc, asum, amax, acnt,
             rowbuf, meanbuf, maxbuf, xdbuf, sem_i, sem_r):
  wid = lax.axis_index("s") * NC + lax.axis_index("c")
  lo = wid * RPT
  nrows = jnp.minimum(N_DST - lo, RPT)
  nrows_u = nrows.astype(jnp.uint32)
  zi = jnp.zeros((L,), jnp.int32)
  zf = jnp.zeros((L,), jnp.float32)
  ninf = jnp.full((L,), -jnp.inf, jnp.float32)
  ones = jnp.ones((L,), jnp.float32)
  trashv = jnp.full((L,), TRASH, jnp.int32)

  def init_row(r, c):
    for f in range(FG):
      asum[r, pl.ds(f * L, L)] = zf
      amax[r, pl.ds(f * L, L)] = ninf
    return c
  lax.fori_loop(0, ROWS, init_row, 0)

  def init_cnt(i, c):
    acnt[pl.ds(i * L, L)] = zf
    return c
  lax.fori_loop(0, ROWS // L, init_cnt, 0)

  def init_msrc(i, c):
    msrc[pl.ds(i * L, L)] = zi
    return c
  lax.fori_loop(0, (CH + SB * L) // L, init_msrc, 0)

  def fire_idx(c, b):
    pltpu.async_copy(dsti.at[pl.ds(c * CH, CH)], dstbuf.at[b], sem_i)
    pltpu.async_copy(srci.at[pl.ds(c * CH, CH)], srcbuf.at[b], sem_i)

  def wait_idx(b):
    pltpu.make_async_copy(dsti.at[pl.ds(0, CH)], dstbuf.at[b], sem_i).wait()
    pltpu.make_async_copy(srci.at[pl.ds(0, CH)], srcbuf.at[b], sem_i).wait()

  def fire_g(g, gb):
    idxv = msrc[pl.ds(g * L, L)]
    pltpu.async_copy(xs.at[idxv], rowbuf.at[gb], sem_r)

  def drain_g(gb):
    pltpu.make_async_copy(xs.at[zi], rowbuf.at[gb], sem_r).wait()

  fire_idx(0, 0)

  def chunk_body(c, carry):
    cb = c & 1
    wait_idx(cb)

    @pl.when(c + 1 < NCH)
    def _():
      fire_idx(c + 1, 1 - cb)

    @plsc.parallel_loop(0, CH // L, unroll=4, carry=jnp.int32(0))
    def scan_loop(i, n):
      dvec = dstbuf[cb, pl.ds(i * L, L)]
      svec = srcbuf[cb, pl.ds(i * L, L)]
      m = (dvec >= lo) & (dvec < lo + nrows)
      cs = plsc.cumsum(m.astype(jnp.int32))
      pos = n + cs - 1
      plsc.store_scatter(midx, [pos], dvec - lo, mask=m)
      plsc.store_scatter(msrc, [pos], svec, mask=m)
      return n + cs[L - 1]

    n = scan_loop
    midx[pl.ds(n, L)] = trashv
    ng = (n + L - 1) // L

    @pl.when(ng > 0)
    def _():
      fire_g(0, 0)

    def g_body(g, carry2):
      gb = g & 1
      drain_g(gb)

      @pl.when(g + 1 < ng)
      def _():
        fire_g(g + 1, 1 - gb)

      dvec = midx[pl.ds(g * L, L)]
      plsc.addupdate_scatter(acnt, [dvec], ones)
      for j in range(L):
        d = dvec[j]
        for f in range(FG):
          v = rowbuf[gb, j, pl.ds(f * L, L)]
          plsc.addupdate(asum.at[d, pl.ds(f * L, L)], v)
          amax[d, pl.ds(f * L, L)] = jnp.maximum(
              amax[d, pl.ds(f * L, L)], v)
      return carry2

    lax.fori_loop(0, ng, g_body, 0)
    return carry

  lax.fori_loop(0, NCH, chunk_body, 0)

  def fin_body(b, carry):
    r0 = b * L
    cvec = acnt[pl.ds(r0, L)]
    rvec = 1.0 / jnp.maximum(cvec, 1.0)
    pltpu.sync_copy(xd.at[pl.ds(lo + r0, L)], xdbuf)
    for j in range(L):
      cj = cvec[j]
      rj = rvec[j]
      for f in range(FG):
        s = asum[r0 + j, pl.ds(f * L, L)]
        meanbuf[j, pl.ds(f * L, L)] = s * rj
        mx = amax[r0 + j, pl.ds(f * L, L)]
        maxbuf[j, pl.ds(f * L, L)] = jnp.where(cj > 0.0, mx, zf)
    pltpu.sync_copy(xdbuf, out.at[pl.ds(lo + r0, L), pl.ds(0, D)])
    pltpu.sync_copy(meanbuf, out.at[pl.ds(lo + r0, L), pl.ds(D, D)])
    pltpu.sync_copy(maxbuf, out.at[pl.ds(lo + r0, L), pl.ds(2 * D, D)])
    return carry

  lax.fori_loop(0, nrows // L, fin_body, 0)


_sc_kernel = functools.partial(
    pl.kernel,
    out_type=jax.ShapeDtypeStruct((N_DST, 3 * D), jnp.float32),
    mesh=plsc.VectorSubcoreMesh(
        core_axis_name="c", subcore_axis_name="s",
        num_cores=NC, num_subcores=NS),
    compiler_params=pltpu.CompilerParams(needs_layout_passes=False),
    scratch_types=[
        pltpu.VMEM((2, CH), jnp.int32),
        pltpu.VMEM((2, CH), jnp.int32),
        pltpu.VMEM((CH + SB * L,), jnp.int32),
        pltpu.VMEM((CH + SB * L,), jnp.int32),
        pltpu.VMEM((ROWS, D), jnp.float32),
        pltpu.VMEM((ROWS, D), jnp.float32),
        pltpu.VMEM((ROWS,), jnp.float32),
        pltpu.VMEM((2, L, D), jnp.float32),
        pltpu.VMEM((L, D), jnp.float32),
        pltpu.VMEM((L, D), jnp.float32),
        pltpu.VMEM((L, D), jnp.float32),
        pltpu.SemaphoreType.DMA,
        pltpu.SemaphoreType.DMA,
    ],
)(_sc_body)


def kernel(x_src, x_dst, edge_index):
  return _sc_kernel(x_src, x_dst, edge_index[0], edge_index[1])
